# Initial kernel scaffold; baseline (speedup 1.0000x reference)
import jax
import jax.numpy as jnp
from jax.experimental import pallas as pl


def kernel(x, y, W_experts, b_experts, gate_W, gate_b):
    raise NotImplementedError("write your pallas kernel here")


# speedup vs baseline: 2.7861x; 2.7861x over previous
import functools

import jax
import jax.numpy as jnp
from jax import lax
from jax.experimental import pallas as pl
from jax.experimental.pallas import tpu as pltpu
from jax.experimental.pallas import tpu_sc as plsc

D = 2048
LAT = 1024
HP = D // 2
E = 8
K = 2
B = 8192
BM = 256
P = B * K + E * BM
G = P // BM
GATE_BM = 512

NC, NS = 2, 16
NW = NC * NS
TPW = B // NW
DT = 32
CT = 8
NDCH = TPW // DT
NCCH = TPW // CT
CGRP = HP // 16
CUNR = 4


def _gate_body(x_ref, y_ref, gw_ref, gb_ref, idx_ref, gates_ref, xp_ref):
    x = x_ref[...]
    y = y_ref[...]
    xp_ref[...] = pltpu.pack_elementwise([x, y], packed_dtype=jnp.bfloat16)
    gw = gw_ref[...]
    logits = lax.dot_general(x, gw[:, :LAT], (((1,), (1,)), ((), ())),
                             preferred_element_type=jnp.float32)
    logits = logits + lax.dot_general(y, gw[:, LAT:],
                                      (((1,), (1,)), ((), ())),
                                      preferred_element_type=jnp.float32)
    logits = logits + gb_ref[...]
    ecol = lax.broadcasted_iota(jnp.int32, logits.shape, 1)
    m1 = jnp.max(logits, axis=1, keepdims=True)
    i1 = jnp.min(jnp.where(logits == m1, ecol, E), axis=1, keepdims=True)
    masked = jnp.where(ecol == i1, -jnp.inf, logits)
    m2 = jnp.max(masked, axis=1, keepdims=True)
    i2 = jnp.min(jnp.where(masked == m2, ecol, E), axis=1, keepdims=True)
    e2 = jnp.exp(m2 - m1)
    g0 = 1.0 / (1.0 + e2)
    idx_ref[...] = jnp.concatenate([i1, i2], axis=1)
    gates_ref[...] = jnp.concatenate([g0, 1.0 - g0], axis=1)


def _gemm_body(be_ref, x_ref, w_ref, b_ref, y_ref):
    xp = x_ref[...]
    xa = pltpu.unpack_elementwise(
        xp, index=0, packed_dtype=jnp.bfloat16, unpacked_dtype=jnp.float32)
    xb = pltpu.unpack_elementwise(
        xp, index=1, packed_dtype=jnp.bfloat16, unpacked_dtype=jnp.float32)
    w = w_ref[0]
    acc = lax.dot_general(xa, w[:, :HP], (((1,), (1,)), ((), ())),
                          preferred_element_type=jnp.float32)
    acc = acc + lax.dot_general(xb, w[:, HP:], (((1,), (1,)), ((), ())),
                                preferred_element_type=jnp.float32)
    z = acc + b_ref[0]
    y_ref[...] = pltpu.pack_elementwise(
        [z[:, :HP], z[:, HP:]], packed_dtype=jnp.bfloat16)


def _dispatch_body(xp_hbm, pos0_hbm, pos1_hbm, xd_hbm,
                   rows_a, i0a, i1a, rows_b, i0b, i1b,
                   sl_a, ss_a, sl_b, ss_b):
    wid = lax.axis_index("s") * NC + lax.axis_index("c")
    base = wid * TPW
    slots = ((rows_a, i0a, i1a, sl_a, ss_a), (rows_b, i0b, i1b, sl_b, ss_b))

    def start(c, slot):
        rows, idx0, idx1, sl, ss = slot
        tb = base + c * DT
        pltpu.async_copy(xp_hbm.at[pl.ds(tb, DT)], rows, sl)
        pltpu.sync_copy(pos0_hbm.at[pl.ds(tb, DT)], idx0)
        pltpu.sync_copy(pos1_hbm.at[pl.ds(tb, DT)], idx1)

    def finish(c, slot):
        rows, idx0, idx1, sl, ss = slot
        tb = base + c * DT
        pltpu.make_async_copy(xp_hbm.at[pl.ds(tb, DT)], rows, sl).wait()
        pltpu.async_copy(rows, xd_hbm.at[idx0], ss)
        pltpu.async_copy(rows, xd_hbm.at[idx1], ss)

    def wait_scatters(slot):
        rows, idx0, idx1, sl, ss = slot
        pltpu.make_async_copy(rows, xd_hbm.at[idx0], ss).wait()
        pltpu.make_async_copy(rows, xd_hbm.at[idx1], ss).wait()

    start(0, slots[0])

    def body(i, carry):
        for b2 in range(2):
            c = i * 2 + b2
            nxt = slots[(b2 + 1) % 2]

            @pl.when(c + 1 < NDCH)
            def _():
                @pl.when(c >= 1)
                def _():
                    wait_scatters(nxt)
                start(c + 1, nxt)

            finish(c, slots[b2])
        return carry

    lax.fori_loop(0, NDCH // 2, body, 0)
    for c in (NDCH - 2, NDCH - 1):
        wait_scatters(slots[c % 2])


def _combine_body(yd_hbm, pos0_hbm, pos1_hbm, g0_hbm, g1_hbm, out_hbm,
                  b0a, b1a, oa, i0a, i1a, g0a, g1a,
                  b0b, b1b, ob, i0b, i1b, g0b, g1b,
                  sg_a, ss_a, sg_b, ss_b):
    wid = lax.axis_index("s") * NC + lax.axis_index("c")
    base = wid * TPW
    slots = ((b0a, b1a, oa, i0a, i1a, g0a, g1a, sg_a, ss_a),
             (b0b, b1b, ob, i0b, i1b, g0b, g1b, sg_b, ss_b))

    def start(c, slot):
        buf0, buf1, obuf, idx0, idx1, ga, gb, sg, ss = slot
        tb = base + c * CT
        pltpu.sync_copy(pos0_hbm.at[pl.ds(tb, CT)], idx0)
        pltpu.sync_copy(pos1_hbm.at[pl.ds(tb, CT)], idx1)
        pltpu.sync_copy(g0_hbm.at[pl.ds(tb, CT)], ga.at[pl.ds(0, CT)])
        pltpu.sync_copy(g1_hbm.at[pl.ds(tb, CT)], gb.at[pl.ds(0, CT)])
        pltpu.async_copy(yd_hbm.at[idx0], buf0, sg)
        pltpu.async_copy(yd_hbm.at[idx1], buf1, sg)

    def wait_store(c, slot):
        obuf = slot[2]
        ss = slot[8]
        tb = base + c * CT
        pltpu.make_async_copy(obuf, out_hbm.at[pl.ds(tb, CT)], ss).wait()

    def finish(c, slot):
        buf0, buf1, obuf, idx0, idx1, ga, gb, sg, ss = slot
        tb = base + c * CT
        pltpu.make_async_copy(yd_hbm.at[idx0], buf0, sg).wait()
        pltpu.make_async_copy(yd_hbm.at[idx1], buf1, sg).wait()
        gav = ga[...]
        gbv = gb[...]
        for t in range(CT):
            g0v = gav[t]
            g1v = gbv[t]

            def inner(j, c2):
                for u in range(CUNR):
                    g = j * CUNR + u
                    sl = pl.ds(g * 16, 16)
                    v0 = buf0[t, sl]
                    v1 = buf1[t, sl]
                    a0 = lax.bitcast_convert_type(lax.shift_left(v0, 16), jnp.float32)
                    a1 = lax.bitcast_convert_type(lax.shift_left(v1, 16), jnp.float32)
                    b0 = lax.bitcast_convert_type(v0 & jnp.int32(-65536), jnp.float32)
                    b1 = lax.bitcast_convert_type(v1 & jnp.int32(-65536), jnp.float32)
                    obuf[t, sl] = g0v * a0 + g1v * a1
                    obuf[t, pl.ds(HP + g * 16, 16)] = g0v * b0 + g1v * b1
                return c2

            lax.fori_loop(0, CGRP // CUNR, inner, 0)
        pltpu.async_copy(obuf, out_hbm.at[pl.ds(tb, CT)], ss)

    start(0, slots[0])

    def body(i, carry):
        for b2 in range(2):
            c = i * 2 + b2
            nxt = slots[(b2 + 1) % 2]

            @pl.when(c + 1 < NCCH)
            def _():
                @pl.when(c >= 1)
                def _():
                    wait_store(c - 1, nxt)
                start(c + 1, nxt)

            finish(c, slots[b2])
        return carry

    lax.fori_loop(0, NCCH // 2, body, 0)
    for c in (NCCH - 2, NCCH - 1):
        wait_store(c, slots[c % 2])


@functools.cache
def _sc_kernels():
    mesh = plsc.VectorSubcoreMesh(
        core_axis_name="c", subcore_axis_name="s",
        num_cores=NC, num_subcores=NS)
    dispatch = pl.kernel(
        _dispatch_body,
        out_type=jax.ShapeDtypeStruct((P, HP), jnp.int32),
        mesh=mesh,
        scratch_types=[
            pltpu.VMEM((DT, HP), jnp.int32),
            pltpu.VMEM((DT,), jnp.int32),
            pltpu.VMEM((DT,), jnp.int32),
            pltpu.VMEM((DT, HP), jnp.int32),
            pltpu.VMEM((DT,), jnp.int32),
            pltpu.VMEM((DT,), jnp.int32),
            pltpu.SemaphoreType.DMA,
            pltpu.SemaphoreType.DMA,
            pltpu.SemaphoreType.DMA,
            pltpu.SemaphoreType.DMA,
        ],
    )
    combine_scratch = [
        pltpu.VMEM((CT, HP), jnp.int32),
        pltpu.VMEM((CT, HP), jnp.int32),
        pltpu.VMEM((CT, D), jnp.float32),
        pltpu.VMEM((CT,), jnp.int32),
        pltpu.VMEM((CT,), jnp.int32),
        pltpu.VMEM((16,), jnp.float32),
        pltpu.VMEM((16,), jnp.float32),
    ]
    combine = pl.kernel(
        _combine_body,
        out_type=jax.ShapeDtypeStruct((B, D), jnp.float32),
        mesh=mesh,
        scratch_types=combine_scratch + combine_scratch + [
            pltpu.SemaphoreType.DMA,
            pltpu.SemaphoreType.DMA,
            pltpu.SemaphoreType.DMA,
            pltpu.SemaphoreType.DMA,
        ],
    )
    return dispatch, combine


def _routing(top_idx):
    e_flat = top_idx.reshape(-1)
    oneh = (e_flat[:, None] == jnp.arange(E, dtype=jnp.int32)).astype(jnp.int32)
    excl = jnp.cumsum(oneh, axis=0) - oneh
    rank = jnp.take_along_axis(excl, e_flat[:, None], axis=1)[:, 0]
    counts = jnp.sum(oneh, axis=0)
    padded = ((counts + BM - 1) // BM) * BM
    starts = jnp.concatenate(
        [jnp.zeros((1,), jnp.int32), jnp.cumsum(padded)[:-1].astype(jnp.int32)])
    pos = starts[e_flat] + rank
    bend = jnp.cumsum(padded // BM)
    block_expert = jnp.minimum(
        jnp.sum((jnp.arange(G, dtype=jnp.int32)[:, None] >= bend[None, :])
                .astype(jnp.int32), axis=1), E - 1).astype(jnp.int32)
    return pos[0::2], pos[1::2], block_expert


def kernel(x, y, W_experts, b_experts, gate_W, gate_b):
    top_idx, gates, xp = pl.pallas_call(
        _gate_body,
        grid=(B // GATE_BM,),
        in_specs=[
            pl.BlockSpec((GATE_BM, LAT), lambda i: (i, 0)),
            pl.BlockSpec((GATE_BM, LAT), lambda i: (i, 0)),
            pl.BlockSpec((E, D), lambda i: (0, 0)),
            pl.BlockSpec((1, E), lambda i: (0, 0)),
        ],
        out_specs=[
            pl.BlockSpec((GATE_BM, K), lambda i: (i, 0)),
            pl.BlockSpec((GATE_BM, K), lambda i: (i, 0)),
            pl.BlockSpec((GATE_BM, HP), lambda i: (i, 0)),
        ],
        out_shape=[
            jax.ShapeDtypeStruct((B, K), jnp.int32),
            jax.ShapeDtypeStruct((B, K), jnp.float32),
            jax.ShapeDtypeStruct((B, HP), jnp.int32),
        ],
    )(x, y, gate_W, gate_b.reshape(1, E))

    pos0, pos1, block_expert = _routing(top_idx)
    g0 = gates[:, 0]
    g1 = gates[:, 1]

    dispatch, combine = _sc_kernels()
    x_disp = dispatch(xp, pos0, pos1)

    y_disp = pl.pallas_call(
        _gemm_body,
        grid_spec=pltpu.PrefetchScalarGridSpec(
            num_scalar_prefetch=1,
            grid=(G,),
            in_specs=[
                pl.BlockSpec((BM, HP), lambda i, be: (i, 0)),
                pl.BlockSpec((1, D, D), lambda i, be: (be[i], 0, 0)),
                pl.BlockSpec((1, 1, D), lambda i, be: (be[i], 0, 0)),
            ],
            out_specs=pl.BlockSpec((BM, HP), lambda i, be: (i, 0)),
        ),
        out_shape=jax.ShapeDtypeStruct((P, HP), jnp.int32),
    )(block_expert, x_disp, W_experts, b_experts.reshape(E, 1, D))

    return combine(y_disp, pos0, pos1, g0, g1)
